# baseline (speedup 1.0000x reference)
import functools

import jax
import jax.numpy as jnp
from jax import lax
from jax.experimental import pallas as pl
from jax.experimental.pallas import tpu as pltpu
from jax.experimental.pallas import tpu_sc as plsc

N, E, D, G, OUT = 10000, 640000, 128, 64, 2
NP = 10240
NC, NS, L = 2, 16, 16
NW = NC * NS
NPT = NP // NS
XB = 2048
BN_SCALE = 1.0 / (1.0 + 1e-5) ** 0.5

BCH = 156
NLONG_AT = 24
EPTM = 157 * 128
EHA = 79 * 128
EHB = 78 * 128


def _rsqrt16(d):
    i = plsc.bitcast(d, jnp.int32)
    i = jnp.full((L,), 0x5F3759DF, jnp.int32) - (i >> 1)
    y = plsc.bitcast(i, jnp.float32)
    for _ in range(3):
        y = y * (1.5 - 0.5 * d * y * y)
    return y


def _tile_window(cid, sid):
    tile = cid * NS + sid
    extra = jnp.maximum(tile - NLONG_AT, 0)
    base = (BCH * tile + extra) * 128
    return base, tile < NLONG_AT


def _deg_body(ei_hbm, out_hbm, eidx_v, idxd_v, ones_v, zb_v, deg_sh, sem):
    cid = lax.axis_index("c")
    sid = lax.axis_index("s")
    base, is_short = _tile_window(cid, sid)

    cp = pltpu.async_copy(ei_hbm.at[:, pl.ds(base, EPTM)], eidx_v, sem)

    @pl.loop(0, NPT // L)
    def _(j):
        zb_v[pl.ds(j * L, L)] = jnp.zeros((L,), jnp.float32)

    pltpu.sync_copy(zb_v, deg_sh.at[pl.ds(sid * NPT, NPT)])
    cp.wait()

    @pl.loop(0, EPTM // L)
    def _(j):
        sl = pl.ds(j * L, L)
        ones_v[sl] = jnp.full((L,), 1.0, jnp.float32)
        idxd_v[sl] = eidx_v[1, sl]

    @pl.when(is_short)
    def _():
        for j in range(128 // L):
            ones_v[pl.ds(EPTM - 128 + j * L, L)] = jnp.zeros((L,), jnp.float32)

    plsc.subcore_barrier()
    pltpu.sync_copy(ones_v, deg_sh.at[idxd_v], add=True)
    plsc.subcore_barrier()
    pltpu.sync_copy(deg_sh.at[pl.ds(sid * NPT, NPT)],
                    out_hbm.at[cid, pl.ds(sid * NPT, NPT)])


def _agg_body(ei_hbm, h_hbm, degp_hbm, out_hbm, g_hbm,
              scrA, scrB, idxdA, idxdB, valsA, valsB, g_loc,
              d0_v, d1_v, h_v, g_v, zb_v, agg_sh, semA, semB, semG):
    cid = lax.axis_index("c")
    sid = lax.axis_index("s")
    base, is_short = _tile_window(cid, sid)
    nbase = sid * NPT

    cpA = pltpu.async_copy(ei_hbm.at[:, pl.ds(base, EHA)], scrA, semA)
    cpB = pltpu.async_copy(ei_hbm.at[:, pl.ds(base + EHA, EHB)], scrB,
                           semB)

    @pl.loop(0, NPT // L)
    def _(j):
        zb_v[pl.ds(j * L, L)] = jnp.zeros((L,), jnp.float32)

    pltpu.sync_copy(zb_v, agg_sh.at[pl.ds(nbase, NPT)])

    pltpu.sync_copy(degp_hbm.at[0, pl.ds(nbase, NPT)], d0_v)
    pltpu.sync_copy(degp_hbm.at[1, pl.ds(nbase, NPT)], d1_v)
    pltpu.sync_copy(h_hbm.at[pl.ds(nbase, NPT)], h_v)

    @pl.loop(0, NPT // L)
    def _(j):
        sl = pl.ds(j * L, L)
        d = d0_v[sl] + d1_v[sl] + 1.0
        g_v[sl] = h_v[sl] * _rsqrt16(d)

    pltpu.sync_copy(g_v, g_hbm.at[cid, pl.ds(nbase, NPT)])
    cpA.wait()
    cpB.wait()
    plsc.subcore_barrier()
    cpG = pltpu.async_copy(g_hbm.at[cid], g_loc, semG)

    @pl.loop(0, EHA // L)
    def _(j):
        sl = pl.ds(j * L, L)
        idxdA[sl] = scrA[1, sl]

    @pl.loop(0, EHB // L)
    def _(j):
        sl = pl.ds(j * L, L)
        idxdB[sl] = scrB[1, sl]

    cpG.wait()

    @pl.loop(0, EHA // L)
    def _(j):
        sl = pl.ds(j * L, L)
        valsA[sl] = plsc.load_gather(g_loc, [scrA[0, sl]])

    scatA = pltpu.async_copy(valsA, agg_sh.at[idxdA], semA, add=True)

    @pl.loop(0, EHB // L)
    def _(j):
        sl = pl.ds(j * L, L)
        valsB[sl] = plsc.load_gather(g_loc, [scrB[0, sl]])

    @pl.when(is_short)
    def _():
        for j in range(128 // L):
            valsB[pl.ds(EHB - 128 + j * L, L)] = jnp.zeros((L,), jnp.float32)

    scatB = pltpu.async_copy(valsB, agg_sh.at[idxdB], semB, add=True)
    scatA.wait()
    scatB.wait()
    plsc.subcore_barrier()
    pltpu.sync_copy(agg_sh.at[pl.ds(nbase, NPT)],
                    out_hbm.at[cid, pl.ds(nbase, NPT)])


@functools.lru_cache(maxsize=1)
def _sc_kernels():
    mesh = plsc.VectorSubcoreMesh(core_axis_name="c", subcore_axis_name="s",
                                  num_cores=NC, num_subcores=NS)
    params = pltpu.CompilerParams(needs_layout_passes=False)
    deg_kernel = pl.kernel(
        _deg_body,
        compiler_params=params,
        out_type=jax.ShapeDtypeStruct((NC, NP), jnp.float32),
        mesh=mesh,
        scratch_types=[
            pltpu.VMEM((2, EPTM), jnp.int32),
            pltpu.VMEM((EPTM,), jnp.int32),
            pltpu.VMEM((EPTM,), jnp.float32),
            pltpu.VMEM((NPT,), jnp.float32),
            pltpu.VMEM_SHARED((NP,), jnp.float32),
            pltpu.SemaphoreType.DMA,
        ],
    )
    agg_kernel = pl.kernel(
        _agg_body,
        compiler_params=params,
        out_type=[jax.ShapeDtypeStruct((NC, NP), jnp.float32),
                  jax.ShapeDtypeStruct((NC, NP), jnp.float32)],
        mesh=mesh,
        scratch_types=[
            pltpu.VMEM((2, EHA), jnp.int32),
            pltpu.VMEM((2, EHB), jnp.int32),
            pltpu.VMEM((EHA,), jnp.int32),
            pltpu.VMEM((EHB,), jnp.int32),
            pltpu.VMEM((EHA,), jnp.float32),
            pltpu.VMEM((EHB,), jnp.float32),
            pltpu.VMEM((NP,), jnp.float32),
            pltpu.VMEM((NPT,), jnp.float32),
            pltpu.VMEM((NPT,), jnp.float32),
            pltpu.VMEM((NPT,), jnp.float32),
            pltpu.VMEM((NPT,), jnp.float32),
            pltpu.VMEM((NPT,), jnp.float32),
            pltpu.VMEM_SHARED((NP,), jnp.float32),
            pltpu.SemaphoreType.DMA,
            pltpu.SemaphoreType.DMA,
            pltpu.SemaphoreType.DMA,
        ],
    )
    return deg_kernel, agg_kernel


def _c1_body(x_ref, w_ref, h_ref):
    i = pl.program_id(0)
    h = jnp.dot(x_ref[...], w_ref[...], preferred_element_type=jnp.float32)[:, 0]
    row = i * XB + lax.broadcasted_iota(jnp.int32, (XB,), 0)
    h_ref[...] = jnp.where(row < N, h, 0.0)


_c1_call = pl.pallas_call(
    _c1_body,
    grid=(NP // XB,),
    in_specs=[
        pl.BlockSpec((XB, D), lambda i: (i, 0)),
        pl.BlockSpec((D, 1), lambda i: (0, 0)),
    ],
    out_specs=pl.BlockSpec((XB,), lambda i: (i,)),
    out_shape=jax.ShapeDtypeStruct((NP,), jnp.float32),
)


def _e_body(degp_ref, h_ref, aggp_ref, batch_ref, bgcn_ref, w1_ref, b1_ref,
            gam_ref, bet_ref, w2_ref, b2_ref, out_ref):
    deg = degp_ref[0, :] + degp_ref[1, :] + 1.0
    dinv = lax.rsqrt(deg)
    g = h_ref[...] * dinv
    s = aggp_ref[0, :] + aggp_ref[1, :]
    z = jnp.maximum(dinv * (s + g) + bgcn_ref[0], 0.0)
    grp = lax.broadcasted_iota(jnp.int32, (NP, G), 1)
    m = (batch_ref[...][:, None] == grp).astype(jnp.float32)
    sums = jnp.dot(z[None, :], m, preferred_element_type=jnp.float32)[0]
    counts = jnp.sum(m, axis=0)
    pooled = sums / jnp.maximum(counts, 1.0)
    t = pooled * w1_ref[0, 0] + b1_ref[0]
    t = t * (gam_ref[0] * BN_SCALE) + bet_ref[0]
    t = jnp.maximum(t, 0.0)
    o = t[:, None] * w2_ref[...] + b2_ref[...][None, :]
    out_ref[...] = jax.nn.sigmoid(o)


_e_call = pl.pallas_call(
    _e_body,
    out_shape=jax.ShapeDtypeStruct((G, OUT), jnp.float32),
)


def kernel(x, edge_index, batch, W_gcn, b_gcn, W1, b1, bn_gamma, bn_beta, W2, b2):
    batch_pad = jnp.concatenate([batch, jnp.full((NP - N,), G + 63, jnp.int32)])
    deg_kernel, agg_kernel = _sc_kernels()
    h = _c1_call(x, W_gcn)
    degp = deg_kernel(edge_index)
    aggp, _ = agg_kernel(edge_index, h, degp)
    return _e_call(degp, h, aggp, batch_pad, b_gcn, W1, b1, bn_gamma,
                   bn_beta, W2, b2)


# speedup vs baseline: 1.0340x; 1.0340x over previous
import functools

import jax
import jax.numpy as jnp
from jax import lax
from jax.experimental import pallas as pl
from jax.experimental.pallas import tpu as pltpu
from jax.experimental.pallas import tpu_sc as plsc

N, E, D, G, OUT = 10000, 640000, 128, 64, 2
NP = 10240
NC, NS, L = 2, 16, 16
NW = NC * NS
NPT = NP // NS
XB = 2048
BN_SCALE = 1.0 / (1.0 + 1e-5) ** 0.5

BCH = 156
NLONG_AT = 24
EPTM = 157 * 128
EHA = 79 * 128
EHB = 78 * 128


def _rsqrt16(d):
    i = plsc.bitcast(d, jnp.int32)
    i = jnp.full((L,), 0x5F3759DF, jnp.int32) - (i >> 1)
    y = plsc.bitcast(i, jnp.float32)
    for _ in range(3):
        y = y * (1.5 - 0.5 * d * y * y)
    return y


def _tile_window(cid, sid):
    tile = cid * NS + sid
    extra = jnp.maximum(tile - NLONG_AT, 0)
    base = (BCH * tile + extra) * 128
    return base, tile < NLONG_AT


def _deg_body(ei_hbm, out_hbm, eidx_v, idxdA, idxdB, ones_v, zb_v, deg_sh,
              semA, semB):
    cid = lax.axis_index("c")
    sid = lax.axis_index("s")
    base, is_short = _tile_window(cid, sid)

    cp = pltpu.async_copy(ei_hbm.at[:, pl.ds(base, EPTM)], eidx_v, semA)

    @pl.loop(0, NPT // L, unroll=8)
    def _(j):
        zb_v[pl.ds(j * L, L)] = jnp.zeros((L,), jnp.float32)

    pltpu.sync_copy(zb_v, deg_sh.at[pl.ds(sid * NPT, NPT)])

    @pl.loop(0, EPTM // L, unroll=8)
    def _(j):
        ones_v[pl.ds(j * L, L)] = jnp.full((L,), 1.0, jnp.float32)

    @pl.when(is_short)
    def _():
        for j in range(128 // L):
            ones_v[pl.ds(EPTM - 128 + j * L, L)] = jnp.zeros((L,), jnp.float32)

    cp.wait()
    plsc.subcore_barrier()

    @pl.loop(0, EHA // L, unroll=8)
    def _(j):
        sl = pl.ds(j * L, L)
        idxdA[sl] = eidx_v[1, sl]

    scatA = pltpu.async_copy(ones_v.at[pl.ds(0, EHA)], deg_sh.at[idxdA],
                             semA, add=True)

    @pl.loop(0, EHB // L, unroll=8)
    def _(j):
        sl = pl.ds(j * L, L)
        idxdB[sl] = eidx_v[1, pl.ds(EHA + j * L, L)]

    scatB = pltpu.async_copy(ones_v.at[pl.ds(EHA, EHB)], deg_sh.at[idxdB],
                             semB, add=True)
    scatA.wait()
    scatB.wait()
    plsc.subcore_barrier()
    pltpu.sync_copy(deg_sh.at[pl.ds(sid * NPT, NPT)],
                    out_hbm.at[cid, pl.ds(sid * NPT, NPT)])


def _agg_body(ei_hbm, h_hbm, degp_hbm, out_hbm, g_hbm,
              scrA, scrB, idxdA, idxdB, valsA, valsB, g_loc,
              d0_v, d1_v, h_v, g_v, zb_v, agg_sh, semA, semB):
    cid = lax.axis_index("c")
    sid = lax.axis_index("s")
    base, is_short = _tile_window(cid, sid)
    nbase = sid * NPT

    cpA = pltpu.async_copy(ei_hbm.at[:, pl.ds(base, EHA)], scrA, semA)
    cpB = pltpu.async_copy(ei_hbm.at[:, pl.ds(base + EHA, EHB)], scrB,
                           semB)

    @pl.loop(0, NPT // L, unroll=8)
    def _(j):
        zb_v[pl.ds(j * L, L)] = jnp.zeros((L,), jnp.float32)

    pltpu.sync_copy(zb_v, agg_sh.at[pl.ds(nbase, NPT)])

    pltpu.sync_copy(degp_hbm.at[0, pl.ds(nbase, NPT)], d0_v)
    pltpu.sync_copy(degp_hbm.at[1, pl.ds(nbase, NPT)], d1_v)
    pltpu.sync_copy(h_hbm.at[pl.ds(nbase, NPT)], h_v)

    @pl.loop(0, NPT // L, unroll=4)
    def _(j):
        sl = pl.ds(j * L, L)
        d = d0_v[sl] + d1_v[sl] + 1.0
        g_v[sl] = h_v[sl] * _rsqrt16(d)

    pltpu.sync_copy(g_v, g_hbm.at[cid, pl.ds(nbase, NPT)])
    cpA.wait()
    cpB.wait()
    plsc.subcore_barrier()
    pltpu.sync_copy(g_hbm.at[cid], g_loc)

    @pl.loop(0, EHA // L, unroll=8)
    def _(j):
        sl = pl.ds(j * L, L)
        valsA[sl] = plsc.load_gather(g_loc, [scrA[0, sl]])
        idxdA[sl] = scrA[1, sl]

    scatA = pltpu.async_copy(valsA, agg_sh.at[idxdA], semA, add=True)

    @pl.loop(0, EHB // L, unroll=8)
    def _(j):
        sl = pl.ds(j * L, L)
        valsB[sl] = plsc.load_gather(g_loc, [scrB[0, sl]])
        idxdB[sl] = scrB[1, sl]

    @pl.when(is_short)
    def _():
        for j in range(128 // L):
            valsB[pl.ds(EHB - 128 + j * L, L)] = jnp.zeros((L,), jnp.float32)

    scatB = pltpu.async_copy(valsB, agg_sh.at[idxdB], semB, add=True)
    scatA.wait()
    scatB.wait()
    plsc.subcore_barrier()
    pltpu.sync_copy(agg_sh.at[pl.ds(nbase, NPT)],
                    out_hbm.at[cid, pl.ds(nbase, NPT)])


@functools.lru_cache(maxsize=1)
def _sc_kernels():
    mesh = plsc.VectorSubcoreMesh(core_axis_name="c", subcore_axis_name="s",
                                  num_cores=NC, num_subcores=NS)
    params = pltpu.CompilerParams(needs_layout_passes=False)
    deg_kernel = pl.kernel(
        _deg_body,
        compiler_params=params,
        out_type=jax.ShapeDtypeStruct((NC, NP), jnp.float32),
        mesh=mesh,
        scratch_types=[
            pltpu.VMEM((2, EPTM), jnp.int32),
            pltpu.VMEM((EHA,), jnp.int32),
            pltpu.VMEM((EHB,), jnp.int32),
            pltpu.VMEM((EPTM,), jnp.float32),
            pltpu.VMEM((NPT,), jnp.float32),
            pltpu.VMEM_SHARED((NP,), jnp.float32),
            pltpu.SemaphoreType.DMA,
            pltpu.SemaphoreType.DMA,
        ],
    )
    agg_kernel = pl.kernel(
        _agg_body,
        compiler_params=params,
        out_type=[jax.ShapeDtypeStruct((NC, NP), jnp.float32),
                  jax.ShapeDtypeStruct((NC, NP), jnp.float32)],
        mesh=mesh,
        scratch_types=[
            pltpu.VMEM((2, EHA), jnp.int32),
            pltpu.VMEM((2, EHB), jnp.int32),
            pltpu.VMEM((EHA,), jnp.int32),
            pltpu.VMEM((EHB,), jnp.int32),
            pltpu.VMEM((EHA,), jnp.float32),
            pltpu.VMEM((EHB,), jnp.float32),
            pltpu.VMEM((NP,), jnp.float32),
            pltpu.VMEM((NPT,), jnp.float32),
            pltpu.VMEM((NPT,), jnp.float32),
            pltpu.VMEM((NPT,), jnp.float32),
            pltpu.VMEM((NPT,), jnp.float32),
            pltpu.VMEM((NPT,), jnp.float32),
            pltpu.VMEM_SHARED((NP,), jnp.float32),
            pltpu.SemaphoreType.DMA,
            pltpu.SemaphoreType.DMA,
        ],
    )
    return deg_kernel, agg_kernel


def _c1_body(x_ref, w_ref, h_ref):
    i = pl.program_id(0)
    h = jnp.dot(x_ref[...], w_ref[...], preferred_element_type=jnp.float32)[:, 0]
    row = i * XB + lax.broadcasted_iota(jnp.int32, (XB,), 0)
    h_ref[...] = jnp.where(row < N, h, 0.0)


_c1_call = pl.pallas_call(
    _c1_body,
    grid=(NP // XB,),
    in_specs=[
        pl.BlockSpec((XB, D), lambda i: (i, 0)),
        pl.BlockSpec((D, 1), lambda i: (0, 0)),
    ],
    out_specs=pl.BlockSpec((XB,), lambda i: (i,)),
    out_shape=jax.ShapeDtypeStruct((NP,), jnp.float32),
)


def _e_body(degp_ref, h_ref, aggp_ref, batch_ref, bgcn_ref, w1_ref, b1_ref,
            gam_ref, bet_ref, w2_ref, b2_ref, out_ref):
    deg = degp_ref[0, :] + degp_ref[1, :] + 1.0
    dinv = lax.rsqrt(deg)
    g = h_ref[...] * dinv
    s = aggp_ref[0, :] + aggp_ref[1, :]
    z = jnp.maximum(dinv * (s + g) + bgcn_ref[0], 0.0)
    grp = lax.broadcasted_iota(jnp.int32, (NP, G), 1)
    m = (batch_ref[...][:, None] == grp).astype(jnp.float32)
    sums = jnp.dot(z[None, :], m, preferred_element_type=jnp.float32)[0]
    counts = jnp.sum(m, axis=0)
    pooled = sums / jnp.maximum(counts, 1.0)
    t = pooled * w1_ref[0, 0] + b1_ref[0]
    t = t * (gam_ref[0] * BN_SCALE) + bet_ref[0]
    t = jnp.maximum(t, 0.0)
    o = t[:, None] * w2_ref[...] + b2_ref[...][None, :]
    out_ref[...] = jax.nn.sigmoid(o)


_e_call = pl.pallas_call(
    _e_body,
    out_shape=jax.ShapeDtypeStruct((G, OUT), jnp.float32),
)


def kernel(x, edge_index, batch, W_gcn, b_gcn, W1, b1, bn_gamma, bn_beta, W2, b2):
    batch_pad = jnp.concatenate([batch, jnp.full((NP - N,), G + 63, jnp.int32)])
    deg_kernel, agg_kernel = _sc_kernels()
    h = _c1_call(x, W_gcn)
    degp = deg_kernel(edge_index)
    aggp, _ = agg_kernel(edge_index, h, degp)
    return _e_call(degp, h, aggp, batch_pad, b_gcn, W1, b1, bn_gamma,
                   bn_beta, W2, b2)
